# Initial kernel scaffold; baseline (speedup 1.0000x reference)
import jax
import jax.numpy as jnp
from jax.experimental import pallas as pl


def kernel(x, edge_index, W_rel_0, W_loop_0, bias_0, W_rel_1, W_loop_1, bias_1):
    raise NotImplementedError("write your pallas kernel here")


# speedup vs baseline: 2.4491x; 2.4491x over previous
import functools

import jax
import jax.numpy as jnp
from jax.experimental import pallas as pl

_N = 10000
_D = 128
_K = 5
_Q = 6
_RB = 1000


def _pca_loc(h):
    hc = h - jnp.mean(h, axis=0, keepdims=True)
    U, S, Vt = jnp.linalg.svd(hc, full_matrices=False)
    return U[:, :_Q]


_QB = 128
_NPAD = 10112


def _knn_body(locq_ref, loc_ref, out_ref):
    i = pl.program_id(0)
    q = locq_ref[...]
    loc = loc_ref[...]
    sq = jnp.sum(loc * loc, axis=1)
    sq_q = jnp.sum(q * q, axis=1)
    prod = jax.lax.dot_general(q, loc, (((1,), (1,)), ((), ())))
    d2 = (sq_q[:, None] - 2.0 * prod) + sq[None, :]
    col = jax.lax.broadcasted_iota(jnp.int32, (_QB, _NPAD), 1)
    rowg = i * _QB + jax.lax.broadcasted_iota(jnp.int32, (_QB, _NPAD), 0)
    d2 = jnp.where(col == rowg, jnp.inf, d2)
    inf = jnp.float32(jnp.inf)
    big = jnp.int32(_NPAD)
    for k in range(_K):
        m = jnp.min(d2, axis=1, keepdims=True)
        idx = jnp.min(jnp.where(d2 == m, col, big), axis=1, keepdims=True)
        out_ref[:, k] = idx[:, 0]
        d2 = jnp.where(col == idx, inf, d2)
    for k in range(_K, 8):
        out_ref[:, k] = jnp.zeros((_QB,), jnp.int32)


def _knn(loc):
    locp = jnp.pad(loc, ((0, _NPAD - _N), (0, 0)), constant_values=1e9)
    locp = jnp.pad(locp, ((0, 0), (0, 8 - _Q)), constant_values=0.0)
    nbr = pl.pallas_call(
        _knn_body,
        grid=(_NPAD // _QB,),
        in_specs=[pl.BlockSpec((_QB, 8), lambda i: (i, 0)),
                  pl.BlockSpec((_NPAD, 8), lambda i: (0, 0))],
        out_specs=pl.BlockSpec((_QB, 8), lambda i: (i, 0)),
        out_shape=jax.ShapeDtypeStruct((_NPAD, 8), jnp.int32),
    )(locp, locp)
    return nbr[:_N, :_K]


def _mm2_body(h_ref, w0_ref, w1_ref, o0_ref, o1_ref):
    h = h_ref[...]
    o0_ref[...] = jnp.dot(h, w0_ref[...])
    o1_ref[...] = jnp.dot(h, w1_ref[...])


def _mm2(h, W0, W1):
    blk = pl.BlockSpec((_RB, _D), lambda i: (i, 0))
    wblk = pl.BlockSpec((_D, _D), lambda i: (0, 0))
    osd = jax.ShapeDtypeStruct((_N, _D), jnp.float32)
    return pl.pallas_call(
        _mm2_body,
        grid=(_N // _RB,),
        in_specs=[blk, wblk, wblk],
        out_specs=[blk, blk],
        out_shape=[osd, osd],
    )(h, W0, W1)


def _combine_body(agg_ref, h_ref, wl_ref, b_ref, o_ref, *, act):
    out = (agg_ref[...] + b_ref[...]) + jnp.dot(h_ref[...], wl_ref[...])
    if act:
        out = jnp.maximum(out, 0.0)
    o_ref[...] = out


def _combine(agg, h, Wl, b, act):
    blk = pl.BlockSpec((_RB, _D), lambda i: (i, 0))
    wblk = pl.BlockSpec((_D, _D), lambda i: (0, 0))
    bblk = pl.BlockSpec((1, _D), lambda i: (0, 0))
    return pl.pallas_call(
        functools.partial(_combine_body, act=act),
        grid=(_N // _RB,),
        in_specs=[blk, blk, wblk, bblk],
        out_specs=blk,
        out_shape=jax.ShapeDtypeStruct((_N, _D), jnp.float32),
    )(agg, h, Wl, b.reshape(1, _D))


def kernel(x, edge_index, W_rel_0, W_loop_0, bias_0, W_rel_1, W_loop_1, bias_1):
    src = edge_index[0]
    dst = edge_index[1]
    params = [(W_rel_0, W_loop_0, bias_0, True), (W_rel_1, W_loop_1, bias_1, False)]
    h = x
    n = h.shape[0]
    for (Wr, Wl, b, act) in params:
        loc = _pca_loc(h)
        nbr = _knn(loc)
        knn_src = nbr.reshape(-1)
        knn_dst = jnp.repeat(jnp.arange(n), _K)
        M0, M1 = _mm2(h, Wr[0], Wr[1])
        m0 = M0[src]
        m1 = M1[knn_src]
        agg = jnp.zeros((_N, _D), jnp.float32).at[dst].add(m0)
        agg = agg.at[knn_dst].add(m1)
        h = _combine(agg, h, Wl, b, act)
    return h


# speedup vs baseline: 3.2855x; 1.3415x over previous
import functools

import jax
import jax.numpy as jnp
from jax import lax
from jax.experimental import pallas as pl
from jax.experimental.pallas import tpu as pltpu
from jax.experimental.pallas import tpu_sc as plsc

_N = 10000
_D = 128
_K = 5
_Q = 6
_RB = 1000


def _pca_loc(h):
    hc = h - jnp.mean(h, axis=0, keepdims=True)
    U, S, Vt = jnp.linalg.svd(hc, full_matrices=False)
    return U[:, :_Q]


_QB = 128
_NPAD = 10112


def _knn_body(locq_ref, loc_ref, out_ref):
    i = pl.program_id(0)
    q = locq_ref[...]
    loc = loc_ref[...]
    sq = jnp.sum(loc * loc, axis=1)
    sq_q = jnp.sum(q * q, axis=1)
    prod = jax.lax.dot_general(q, loc, (((1,), (1,)), ((), ())))
    d2 = (sq_q[:, None] - 2.0 * prod) + sq[None, :]
    col = jax.lax.broadcasted_iota(jnp.int32, (_QB, _NPAD), 1)
    rowg = i * _QB + jax.lax.broadcasted_iota(jnp.int32, (_QB, _NPAD), 0)
    d2 = jnp.where(col == rowg, jnp.inf, d2)
    inf = jnp.float32(jnp.inf)
    big = jnp.int32(_NPAD)
    for k in range(_K):
        m = jnp.min(d2, axis=1, keepdims=True)
        idx = jnp.min(jnp.where(d2 == m, col, big), axis=1, keepdims=True)
        out_ref[:, k] = idx[:, 0]
        d2 = jnp.where(col == idx, inf, d2)
    for k in range(_K, 8):
        out_ref[:, k] = jnp.zeros((_QB,), jnp.int32)


def _knn(loc):
    locp = jnp.pad(loc, ((0, _NPAD - _N), (0, 0)), constant_values=1e9)
    locp = jnp.pad(locp, ((0, 0), (0, 8 - _Q)), constant_values=0.0)
    nbr = pl.pallas_call(
        _knn_body,
        grid=(_NPAD // _QB,),
        in_specs=[pl.BlockSpec((_QB, 8), lambda i: (i, 0)),
                  pl.BlockSpec((_NPAD, 8), lambda i: (0, 0))],
        out_specs=pl.BlockSpec((_QB, 8), lambda i: (i, 0)),
        out_shape=jax.ShapeDtypeStruct((_NPAD, 8), jnp.int32),
    )(locp, locp)
    return nbr[:_N, :_K]


def _mm2_body(h_ref, w0_ref, w1_ref, o0_ref, o1_ref):
    h = h_ref[...]
    o0_ref[...] = jnp.dot(h, w0_ref[...])
    o1_ref[...] = jnp.dot(h, w1_ref[...])


def _mm2(h, W0, W1):
    blk = pl.BlockSpec((_RB, _D), lambda i: (i, 0))
    wblk = pl.BlockSpec((_D, _D), lambda i: (0, 0))
    osd = jax.ShapeDtypeStruct((_N, _D), jnp.float32)
    return pl.pallas_call(
        _mm2_body,
        grid=(_N // _RB,),
        in_specs=[blk, wblk, wblk],
        out_specs=[blk, blk],
        out_shape=[osd, osd],
    )(h, W0, W1)


_E = 320000
_NW = 32
_RNG = 320
_NP = _NW * _RNG
_WROWS = 336
_SCH = 2000
_LCAP = 4352
_GB = 128

_sc_mesh = plsc.VectorSubcoreMesh(core_axis_name="c", subcore_axis_name="s")


def _zero_rows(ztile, shared, soff):
    for r in range(16):
        for g in range(8):
            ztile[r, pl.ds(g * 16, 16)] = jnp.zeros((16,), jnp.float32)
    for b in range(_WROWS // 16):
        pltpu.sync_copy(ztile, shared.at[pl.ds(soff + b * 16, 16)])


def _scatter_body(src_hbm, dst_hbm, m0_hbm, out_hbm,
                  srcbuf, dstbuf, slist, dlist, dcur, scur, rows, ztile, shared):
    w = lax.axis_index("s") * 2 + lax.axis_index("c")
    lo = w * _RNG
    soff = (w // 2) * _WROWS
    dump = soff + _RNG
    _zero_rows(ztile, shared, soff)

    def drain_block(bi, _):
        for g in range(_GB // 16):
            dcur[pl.ds(g * 16, 16)] = dlist[pl.ds(bi * _GB + g * 16, 16)]
            scur[pl.ds(g * 16, 16)] = slist[pl.ds(bi * _GB + g * 16, 16)]
        pltpu.sync_copy(m0_hbm.at[scur], rows)
        pltpu.sync_copy(rows, shared.at[dcur], add=True)
        return 0

    def scan_chunk(ci, off):
        base = ci * _SCH
        pltpu.sync_copy(src_hbm.at[pl.ds(base, _SCH)], srcbuf)
        pltpu.sync_copy(dst_hbm.at[pl.ds(base, _SCH)], dstbuf)

        def group(gi, off):
            d = dstbuf[pl.ds(gi * 16, 16)]
            sv = srcbuf[pl.ds(gi * 16, 16)]
            m = (d >= lo) & (d < lo + _RNG)
            dl = (d - lo) + soff
            mi = m.astype(jnp.int32)
            pos = (off + jnp.cumsum(mi)) - mi
            plsc.store_scatter(slist, [pos], sv, mask=m)
            plsc.store_scatter(dlist, [pos], dl, mask=m)
            return off + jnp.sum(mi)

        off = lax.fori_loop(0, _SCH // 16, group, off)
        nfull = off // _GB
        lax.fori_loop(0, nfull, drain_block, 0)

        @pl.when(nfull > 0)
        def _():
            for g in range(_GB // 16):
                slist[pl.ds(g * 16, 16)] = slist[pl.ds(nfull * _GB + g * 16, 16)]
                dlist[pl.ds(g * 16, 16)] = dlist[pl.ds(nfull * _GB + g * 16, 16)]

        return off - nfull * _GB

    off = lax.fori_loop(0, _E // _SCH, scan_chunk, 0)
    zi = jnp.zeros((16,), jnp.int32)
    for g in range(8):
        slist[pl.ds(off + g * 16, 16)] = zi
        dlist[pl.ds(off + g * 16, 16)] = zi + dump

    @pl.when(off > 0)
    def _():
        drain_block(0, 0)

    pltpu.sync_copy(shared.at[pl.ds(soff, _RNG)], out_hbm.at[pl.ds(lo, _RNG)])


def _sc_scatter(src, dst, M0):
    kfn = pl.kernel(
        _scatter_body,
        out_type=jax.ShapeDtypeStruct((_NP, _D), jnp.float32),
        mesh=_sc_mesh,
        compiler_params=pltpu.CompilerParams(needs_layout_passes=False),
        scratch_types=[
            pltpu.VMEM((_SCH,), jnp.int32),
            pltpu.VMEM((_SCH,), jnp.int32),
            pltpu.VMEM((_LCAP,), jnp.int32),
            pltpu.VMEM((_LCAP,), jnp.int32),
            pltpu.VMEM((_GB,), jnp.int32),
            pltpu.VMEM((_GB,), jnp.int32),
            pltpu.VMEM((_GB, _D), jnp.float32),
            pltpu.VMEM((16, _D), jnp.float32),
            pltpu.VMEM_SHARED((16 * _WROWS, _D), jnp.float32),
        ],
    )
    return kfn(src, dst, M0)


def _knnagg_body(agg_hbm, m1_hbm, kidx_hbm, kloc_hbm, out_hbm,
                 icur, lcur, rows, shared):
    w = lax.axis_index("s") * 2 + lax.axis_index("c")
    lo = w * _RNG
    soff = (w // 2) * _WROWS
    pltpu.sync_copy(agg_hbm.at[pl.ds(lo, _RNG)], shared.at[pl.ds(soff, _RNG)])

    def block(bi, _):
        pltpu.sync_copy(kidx_hbm.at[w, pl.ds(bi * _GB, _GB)], icur)
        pltpu.sync_copy(kloc_hbm.at[w, pl.ds(bi * _GB, _GB)], lcur)
        pltpu.sync_copy(m1_hbm.at[icur], rows)
        pltpu.sync_copy(rows, shared.at[lcur], add=True)
        return 0

    lax.fori_loop(0, (_RNG * _K + _GB - 1) // _GB, block, 0)
    pltpu.sync_copy(shared.at[pl.ds(soff, _RNG)], out_hbm.at[pl.ds(lo, _RNG)])


def _sc_knnagg(agg, M1, kidx, kloc):
    kfn = pl.kernel(
        _knnagg_body,
        out_type=jax.ShapeDtypeStruct((_NP, _D), jnp.float32),
        mesh=_sc_mesh,
        compiler_params=pltpu.CompilerParams(needs_layout_passes=False),
        scratch_types=[
            pltpu.VMEM((_GB,), jnp.int32),
            pltpu.VMEM((_GB,), jnp.int32),
            pltpu.VMEM((_GB, _D), jnp.float32),
            pltpu.VMEM_SHARED((16 * _WROWS, _D), jnp.float32),
        ],
    )
    return kfn(agg, M1, kidx, kloc)


_KSEG = -(-(_RNG * _K) // _GB) * _GB


def _knn_lists(nbr):
    nbr_p = jnp.pad(nbr, ((0, _NP - _N), (0, 0)))
    kidx = nbr_p.reshape(_NW, _RNG * _K)
    kidx = jnp.pad(kidx, ((0, 0), (0, _KSEG - _RNG * _K)))
    soff_col = ((jnp.arange(_NW) // 2) * _WROWS)[:, None]
    local = jnp.tile(jnp.repeat(jnp.arange(_RNG), _K)[None, :], (_NW, 1))
    pad = jnp.full((_NW, _KSEG - _RNG * _K), _RNG, jnp.int32)
    kloc = jnp.concatenate([local, pad], axis=1) + soff_col
    return kidx.astype(jnp.int32), kloc.astype(jnp.int32)


def _combine_body(agg_ref, h_ref, wl_ref, b_ref, o_ref, *, act):
    out = (agg_ref[...] + b_ref[...]) + jnp.dot(h_ref[...], wl_ref[...])
    if act:
        out = jnp.maximum(out, 0.0)
    o_ref[...] = out


def _combine(agg, h, Wl, b, act):
    blk = pl.BlockSpec((_RB, _D), lambda i: (i, 0))
    wblk = pl.BlockSpec((_D, _D), lambda i: (0, 0))
    bblk = pl.BlockSpec((1, _D), lambda i: (0, 0))
    return pl.pallas_call(
        functools.partial(_combine_body, act=act),
        grid=(_N // _RB,),
        in_specs=[blk, blk, wblk, bblk],
        out_specs=blk,
        out_shape=jax.ShapeDtypeStruct((_N, _D), jnp.float32),
    )(agg, h, Wl, b.reshape(1, _D))


def kernel(x, edge_index, W_rel_0, W_loop_0, bias_0, W_rel_1, W_loop_1, bias_1):
    src = edge_index[0]
    dst = edge_index[1]
    params = [(W_rel_0, W_loop_0, bias_0, True), (W_rel_1, W_loop_1, bias_1, False)]
    h = x
    for (Wr, Wl, b, act) in params:
        M0, M1 = _mm2(h, Wr[0], Wr[1])
        agg0 = _sc_scatter(src, dst, M0)
        loc = _pca_loc(h)
        nbr = _knn(loc)
        kidx, kloc = _knn_lists(nbr)
        agg = _sc_knnagg(agg0, M1, kidx, kloc)
        h = _combine(agg[:_N], h, Wl, b, act)
    return h
